# baseline (speedup 1.0000x reference)
import functools
import math

import jax
import jax.numpy as jnp
from jax import lax
from jax.experimental import pallas as pl
from jax.experimental.pallas import tpu as pltpu
from jax.experimental.pallas import tpu_sc as plsc

D_MODEL = 64
SCALE = math.sqrt(D_MODEL)

NC = 2
NS = 16
NW = NC * NS

STEP = 128
LANES = 16
CHUNK = 4096


def _make_kernel(n_steps, vocab):
    mesh = plsc.VectorSubcoreMesh(core_axis_name="c", subcore_axis_name="s")

    n_full = vocab // CHUNK
    rem = vocab - n_full * CHUNK
    n_chunks = n_full + (1 if rem else 0)
    t_hi = (n_chunks - 1) // NW + 1

    @functools.partial(
        pl.kernel,
        mesh=mesh,
        compiler_params=pltpu.CompilerParams(needs_layout_passes=False),
        out_type=jax.ShapeDtypeStruct((NW, n_steps, STEP, D_MODEL), jnp.float32),
        scratch_types=[
            pltpu.MemorySpace.HBM((vocab + NW, D_MODEL), jnp.float32),
            pltpu.VMEM((n_steps, STEP), jnp.int32),
            pltpu.VMEM((8, D_MODEL), jnp.float32),
            pltpu.VMEM((1, D_MODEL), jnp.float32),
            pltpu.VMEM((NW, D_MODEL), jnp.float32),
            pltpu.VMEM((STEP, D_MODEL), jnp.float32),
            pltpu.VMEM((STEP, D_MODEL), jnp.float32),
            pltpu.SemaphoreType.DMA,
            pltpu.SemaphoreType.DMA,
            pltpu.SemaphoreType.DMA,
        ],
    )
    def k(x_hbm, tbl_hbm, out_hbm,
          tbl3, idx_v, ebuf, fbuf, pbuf, obuf0, obuf1, psem, gsem, ssem):
        wid = lax.axis_index("s") * NC + lax.axis_index("c")
        obuf = (obuf0, obuf1)

        pltpu.sync_copy(tbl_hbm.at[pl.ds(0, 8)], ebuf)
        expv = ebuf[0, pl.ds(0, LANES)] * 3.0 + 1.337
        for cc in range(D_MODEL // LANES):
            fbuf[0, pl.ds(cc * LANES, LANES)] = expv

        pltpu.sync_copy(x_hbm.at[wid], idx_v)

        def fire(t, c2):
            ci = wid + t * NW

            @pl.when(ci < n_full)
            def _():
                base = ci * CHUNK
                pltpu.async_copy(
                    tbl_hbm.at[pl.ds(base, CHUNK)],
                    tbl3.at[pl.ds(base, CHUNK)], psem)

            if rem:
                @pl.when(ci == n_full)
                def _():
                    base = n_full * CHUNK
                    pltpu.async_copy(
                        tbl_hbm.at[pl.ds(base, rem)],
                        tbl3.at[pl.ds(base, rem)], psem)
            return c2

        def drain(t, c2):
            ci = wid + t * NW

            @pl.when(ci < n_full)
            def _():
                base = ci * CHUNK
                pltpu.make_async_copy(
                    tbl_hbm.at[pl.ds(base, CHUNK)],
                    tbl3.at[pl.ds(base, CHUNK)], psem).wait()

            if rem:
                @pl.when(ci == n_full)
                def _():
                    base = n_full * CHUNK
                    pltpu.make_async_copy(
                        tbl_hbm.at[pl.ds(base, rem)],
                        tbl3.at[pl.ds(base, rem)], psem).wait()
            return c2

        lax.fori_loop(0, t_hi, fire, 0)
        lax.fori_loop(0, t_hi, drain, 0)

        pltpu.sync_copy(fbuf, tbl3.at[pl.ds(vocab + wid, 1)])

        def poll_cond(cnt):
            return cnt < NW * LANES

        def poll_body(cnt):
            pltpu.sync_copy(tbl3.at[pl.ds(vocab, NW)], pbuf)
            c = jnp.int32(0)
            for r in range(NW):
                v = pbuf[r, pl.ds(0, LANES)]
                m = (v == expv).astype(jnp.int32)
                c = c + jnp.sum(m)
            return c

        lax.while_loop(poll_cond, poll_body, jnp.int32(0))

        def scale(buf):
            def row(r, c3):
                for cc in range(D_MODEL // LANES):
                    sl = pl.ds(cc * LANES, LANES)
                    buf[r, sl] = buf[r, sl] * SCALE
                return c3

            lax.fori_loop(0, STEP, row, 0, unroll=4)

        pltpu.async_copy(tbl3.at[idx_v.at[0]], obuf0, gsem)

        def pair(i, carry):
            g0 = i * 2
            for b in range(2):
                g = g0 + b
                nb = 1 - b

                @pl.when(g + 1 < n_steps)
                def _():
                    pltpu.async_copy(
                        tbl3.at[idx_v.at[g + 1]], obuf[nb], gsem)

                pltpu.make_async_copy(
                    tbl3.at[idx_v.at[g]], obuf[b], gsem).wait()

                @pl.when(g >= 2)
                def _():
                    pltpu.make_async_copy(
                        obuf[b], out_hbm.at[wid, g], ssem).wait()

                scale(obuf[b])
                pltpu.async_copy(obuf[b], out_hbm.at[wid, g], ssem)
            return carry

        lax.fori_loop(0, n_steps // 2, pair, 0)

        pltpu.make_async_copy(obuf0, out_hbm.at[wid, 0], ssem).wait()
        pltpu.make_async_copy(obuf1, out_hbm.at[wid, 0], ssem).wait()

    return k


def kernel(x, table):
    b, s = x.shape
    total = b * s
    assert total % (NW * STEP) == 0 and (total // (NW * STEP)) % 2 == 0
    n_steps = total // (NW * STEP)
    v, d = table.shape
    xf = x.reshape(-1).astype(jnp.int32).reshape(NW, n_steps, STEP)
    out = _make_kernel(n_steps, v)(xf, table)
    return out.reshape(b, s, D_MODEL)


# speedup vs baseline: 14.4301x; 14.4301x over previous
import functools
import math

import jax
import jax.numpy as jnp
from jax import lax
from jax.experimental import pallas as pl
from jax.experimental.pallas import tpu as pltpu
from jax.experimental.pallas import tpu_sc as plsc

D_MODEL = 64
SCALE = math.sqrt(D_MODEL)

NC = 2
NS = 16
NW = NC * NS

STEP = 128
LANES = 16
CHUNK = 128


def _make_kernel(n_steps, vocab):
    mesh = plsc.VectorSubcoreMesh(core_axis_name="c", subcore_axis_name="s")

    n_full = vocab // CHUNK
    rem = vocab - n_full * CHUNK
    n_chunks = n_full + (1 if rem else 0)
    t_hi = (n_chunks - 1) // NW + 1

    @functools.partial(
        pl.kernel,
        mesh=mesh,
        compiler_params=pltpu.CompilerParams(needs_layout_passes=False),
        out_type=jax.ShapeDtypeStruct((NW, n_steps, STEP, D_MODEL), jnp.float32),
        scratch_types=[
            pltpu.MemorySpace.HBM((vocab + NW, D_MODEL), jnp.float32),
            pltpu.VMEM((n_steps, STEP), jnp.int32),
            pltpu.VMEM((8, D_MODEL), jnp.float32),
            pltpu.VMEM((1, D_MODEL), jnp.float32),
            pltpu.VMEM((NW, D_MODEL), jnp.float32),
            pltpu.VMEM((STEP, D_MODEL), jnp.float32),
            pltpu.VMEM((STEP, D_MODEL), jnp.float32),
            pltpu.VMEM((CHUNK, D_MODEL), jnp.float32),
            pltpu.VMEM((CHUNK, D_MODEL), jnp.float32),
            pltpu.SemaphoreType.DMA,
            pltpu.SemaphoreType.DMA,
            pltpu.SemaphoreType.DMA,
        ],
    )
    def k(x_hbm, tbl_hbm, out_hbm,
          tbl3, idx_v, ebuf, fbuf, pbuf, obuf0, obuf1, cbuf0, cbuf1,
          psem, gsem, ssem):
        wid = lax.axis_index("s") * NC + lax.axis_index("c")
        obuf = (obuf0, obuf1)

        pltpu.sync_copy(tbl_hbm.at[pl.ds(0, 8)], ebuf)
        expv = ebuf[0, pl.ds(0, LANES)] * 3.0 + 1.337
        for cc in range(D_MODEL // LANES):
            fbuf[0, pl.ds(cc * LANES, LANES)] = expv

        pltpu.sync_copy(x_hbm.at[wid], idx_v)

        cbuf = (cbuf0, cbuf1)

        def p1_read(t, buf):
            ci = wid + t * NW

            @pl.when(ci < n_full)
            def _():
                base = ci * CHUNK
                pltpu.async_copy(tbl_hbm.at[pl.ds(base, CHUNK)], buf, psem)

            if rem:
                @pl.when(ci == n_full)
                def _():
                    base = n_full * CHUNK
                    pltpu.async_copy(
                        tbl_hbm.at[pl.ds(base, rem)],
                        buf.at[pl.ds(0, rem)], psem)

        def p1_wait(t, buf):
            ci = wid + t * NW

            @pl.when(ci < n_full)
            def _():
                base = ci * CHUNK
                pltpu.make_async_copy(
                    tbl_hbm.at[pl.ds(base, CHUNK)], buf, psem).wait()

            if rem:
                @pl.when(ci == n_full)
                def _():
                    base = n_full * CHUNK
                    pltpu.make_async_copy(
                        tbl_hbm.at[pl.ds(base, rem)],
                        buf.at[pl.ds(0, rem)], psem).wait()

        def p1_write(t, buf):
            ci = wid + t * NW

            @pl.when(ci < n_full)
            def _():
                base = ci * CHUNK
                pltpu.sync_copy(buf, tbl3.at[pl.ds(base, CHUNK)])

            if rem:
                @pl.when(ci == n_full)
                def _():
                    base = n_full * CHUNK
                    pltpu.sync_copy(
                        buf.at[pl.ds(0, rem)], tbl3.at[pl.ds(base, rem)])

        p1_read(0, cbuf0)

        def p1_pair(i, c2):
            t0 = i * 2
            for b in range(2):
                t = t0 + b
                nb = 1 - b
                p1_read(t + 1, cbuf[nb])
                p1_wait(t, cbuf[b])
                p1_write(t, cbuf[b])
            return c2

        lax.fori_loop(0, (t_hi + 1) // 2, p1_pair, 0)

        pltpu.sync_copy(fbuf, tbl3.at[pl.ds(vocab + wid, 1)])

        def poll_cond(cnt):
            return cnt < NW * LANES

        def poll_body(cnt):
            pltpu.sync_copy(tbl3.at[pl.ds(vocab, NW)], pbuf)
            c = jnp.int32(0)
            for r in range(NW):
                v = pbuf[r, pl.ds(0, LANES)]
                m = (v == expv).astype(jnp.int32)
                c = c + jnp.sum(m)
            return c

        lax.while_loop(poll_cond, poll_body, jnp.int32(0))

        def scale(buf):
            def row(r, c3):
                for cc in range(D_MODEL // LANES):
                    sl = pl.ds(cc * LANES, LANES)
                    buf[r, sl] = buf[r, sl] * SCALE
                return c3

            lax.fori_loop(0, STEP, row, 0, unroll=4)

        pltpu.async_copy(tbl3.at[idx_v.at[0]], obuf0, gsem)

        def pair(i, carry):
            g0 = i * 2
            for b in range(2):
                g = g0 + b
                nb = 1 - b

                @pl.when(g + 1 < n_steps)
                def _():
                    pltpu.async_copy(
                        tbl3.at[idx_v.at[g + 1]], obuf[nb], gsem)

                pltpu.make_async_copy(
                    tbl3.at[idx_v.at[g]], obuf[b], gsem).wait()

                @pl.when(g >= 2)
                def _():
                    pltpu.make_async_copy(
                        obuf[b], out_hbm.at[wid, g], ssem).wait()

                scale(obuf[b])
                pltpu.async_copy(obuf[b], out_hbm.at[wid, g], ssem)
            return carry

        lax.fori_loop(0, n_steps // 2, pair, 0)

        pltpu.make_async_copy(obuf0, out_hbm.at[wid, 0], ssem).wait()
        pltpu.make_async_copy(obuf1, out_hbm.at[wid, 0], ssem).wait()

    return k


def kernel(x, table):
    b, s = x.shape
    total = b * s
    assert total % (NW * STEP) == 0 and (total // (NW * STEP)) % 2 == 0
    n_steps = total // (NW * STEP)
    v, d = table.shape
    xf = x.reshape(-1).astype(jnp.int32).reshape(NW, n_steps, STEP)
    out = _make_kernel(n_steps, v)(xf, table)
    return out.reshape(b, s, D_MODEL)


# speedup vs baseline: 14.4445x; 1.0010x over previous
import functools
import math

import jax
import jax.numpy as jnp
from jax import lax
from jax.experimental import pallas as pl
from jax.experimental.pallas import tpu as pltpu
from jax.experimental.pallas import tpu_sc as plsc

D_MODEL = 64
SCALE = math.sqrt(D_MODEL)

NC = 2
NS = 16
NW = NC * NS

STEP = 128
LANES = 16
CHUNK = 128


def _make_kernel(n_steps, vocab):
    mesh = plsc.VectorSubcoreMesh(core_axis_name="c", subcore_axis_name="s")

    n_full = vocab // CHUNK
    rem = vocab - n_full * CHUNK
    n_chunks = n_full + (1 if rem else 0)
    t_hi = (n_chunks - 1) // NW + 1

    @functools.partial(
        pl.kernel,
        mesh=mesh,
        compiler_params=pltpu.CompilerParams(needs_layout_passes=False),
        out_type=jax.ShapeDtypeStruct((NW, n_steps, STEP, D_MODEL), jnp.float32),
        scratch_types=[
            pltpu.MemorySpace.HBM((vocab + NW, D_MODEL), jnp.float32),
            pltpu.VMEM((n_steps, STEP), jnp.int32),
            pltpu.VMEM((8, D_MODEL), jnp.float32),
            pltpu.VMEM((1, D_MODEL), jnp.float32),
            pltpu.VMEM((NW, D_MODEL), jnp.float32),
            pltpu.VMEM((STEP, D_MODEL), jnp.float32),
            pltpu.VMEM((STEP, D_MODEL), jnp.float32),
            pltpu.VMEM((CHUNK, D_MODEL), jnp.float32),
            pltpu.VMEM((CHUNK, D_MODEL), jnp.float32),
            pltpu.SemaphoreType.DMA,
            pltpu.SemaphoreType.DMA,
            pltpu.SemaphoreType.DMA,
            pltpu.SemaphoreType.DMA,
            pltpu.SemaphoreType.DMA,
            pltpu.SemaphoreType.DMA,
            pltpu.SemaphoreType.DMA,
            pltpu.SemaphoreType.DMA,
        ],
    )
    def k(x_hbm, tbl_hbm, out_hbm,
          tbl3, idx_v, ebuf, fbuf, pbuf, obuf0, obuf1, cbuf0, cbuf1,
          psem0, psem1, gsem0, gsem1, ssem0, ssem1, wsem0, wsem1):
        wid = lax.axis_index("s") * NC + lax.axis_index("c")
        obuf = (obuf0, obuf1)

        pltpu.sync_copy(tbl_hbm.at[pl.ds(0, 8)], ebuf)
        expv = ebuf[0, pl.ds(0, LANES)] * 3.0 + 1.337
        for cc in range(D_MODEL // LANES):
            fbuf[0, pl.ds(cc * LANES, LANES)] = expv

        pltpu.sync_copy(x_hbm.at[wid], idx_v)

        cbuf = (cbuf0, cbuf1)

        def p1_read(t, buf, sem):
            ci = wid + t * NW

            @pl.when(ci < n_full)
            def _():
                base = ci * CHUNK
                pltpu.async_copy(tbl_hbm.at[pl.ds(base, CHUNK)], buf, sem)

            if rem:
                @pl.when(ci == n_full)
                def _():
                    base = n_full * CHUNK
                    pltpu.async_copy(
                        tbl_hbm.at[pl.ds(base, rem)],
                        buf.at[pl.ds(0, rem)], sem)

        def p1_wait(t, buf, sem):
            ci = wid + t * NW

            @pl.when(ci < n_full)
            def _():
                base = ci * CHUNK
                pltpu.make_async_copy(
                    tbl_hbm.at[pl.ds(base, CHUNK)], buf, sem).wait()

            if rem:
                @pl.when(ci == n_full)
                def _():
                    base = n_full * CHUNK
                    pltpu.make_async_copy(
                        tbl_hbm.at[pl.ds(base, rem)],
                        buf.at[pl.ds(0, rem)], sem).wait()

        def p1_write(t, buf, sem):
            ci = wid + t * NW

            @pl.when(ci < n_full)
            def _():
                base = ci * CHUNK
                pltpu.async_copy(buf, tbl3.at[pl.ds(base, CHUNK)], sem)

            if rem:
                @pl.when(ci == n_full)
                def _():
                    base = n_full * CHUNK
                    pltpu.async_copy(
                        buf.at[pl.ds(0, rem)], tbl3.at[pl.ds(base, rem)], sem)

        def p1_wwait(t, buf, sem):
            ci = wid + t * NW

            @pl.when(ci < n_full)
            def _():
                base = ci * CHUNK
                pltpu.make_async_copy(
                    buf, tbl3.at[pl.ds(base, CHUNK)], sem).wait()

            if rem:
                @pl.when(ci == n_full)
                def _():
                    base = n_full * CHUNK
                    pltpu.make_async_copy(
                        buf.at[pl.ds(0, rem)],
                        tbl3.at[pl.ds(base, rem)], sem).wait()

        psem = (psem0, psem1)
        wsem = (wsem0, wsem1)
        p1_read(0, cbuf0, psem0)

        def p1_pair(i, c2):
            t0 = i * 2
            for b in range(2):
                t = t0 + b
                nb = 1 - b
                @pl.when(t >= 1)
                def _():
                    p1_wwait(t - 1, cbuf[nb], wsem[nb])

                p1_read(t + 1, cbuf[nb], psem[nb])
                p1_wait(t, cbuf[b], psem[b])
                p1_write(t, cbuf[b], wsem[b])
            return c2

        lax.fori_loop(0, (t_hi + 1) // 2, p1_pair, 0)
        if t_hi % 2 == 0:
            p1_wwait(t_hi - 1, cbuf[(t_hi - 1) % 2], wsem[(t_hi - 1) % 2])

        pltpu.sync_copy(fbuf, tbl3.at[pl.ds(vocab + wid, 1)])

        def poll_cond(cnt):
            return cnt < NW * LANES

        def poll_body(cnt):
            pltpu.sync_copy(tbl3.at[pl.ds(vocab, NW)], pbuf)
            c = jnp.int32(0)
            for r in range(NW):
                v = pbuf[r, pl.ds(0, LANES)]
                m = (v == expv).astype(jnp.int32)
                c = c + jnp.sum(m)
            return c

        lax.while_loop(poll_cond, poll_body, jnp.int32(0))

        def scale(buf):
            def row(r, c3):
                for cc in range(D_MODEL // LANES):
                    sl = pl.ds(cc * LANES, LANES)
                    buf[r, sl] = buf[r, sl] * SCALE
                return c3

            lax.fori_loop(0, STEP, row, 0, unroll=4)

        gsem = (gsem0, gsem1)
        ssem = (ssem0, ssem1)
        pltpu.async_copy(tbl3.at[idx_v.at[0]], obuf0, gsem0)

        def pair(i, carry):
            g0 = i * 2
            for b in range(2):
                g = g0 + b
                nb = 1 - b

                @pl.when(g + 1 < n_steps)
                def _():
                    pltpu.async_copy(
                        tbl3.at[idx_v.at[g + 1]], obuf[nb], gsem[nb])

                pltpu.make_async_copy(
                    tbl3.at[idx_v.at[g]], obuf[b], gsem[b]).wait()

                @pl.when(g >= 2)
                def _():
                    pltpu.make_async_copy(
                        obuf[b], out_hbm.at[wid, g], ssem[b]).wait()

                scale(obuf[b])
                pltpu.async_copy(obuf[b], out_hbm.at[wid, g], ssem[b])
            return carry

        lax.fori_loop(0, n_steps // 2, pair, 0)

        pltpu.make_async_copy(obuf0, out_hbm.at[wid, 0], ssem0).wait()
        pltpu.make_async_copy(obuf1, out_hbm.at[wid, 0], ssem1).wait()

    return k


def kernel(x, table):
    b, s = x.shape
    total = b * s
    assert total % (NW * STEP) == 0 and (total // (NW * STEP)) % 2 == 0
    n_steps = total // (NW * STEP)
    v, d = table.shape
    xf = x.reshape(-1).astype(jnp.int32).reshape(NW, n_steps, STEP)
    out = _make_kernel(n_steps, v)(xf, table)
    return out.reshape(b, s, D_MODEL)
